# baseline (speedup 1.0000x reference)
import functools

import jax
import jax.numpy as jnp
from jax import lax
from jax.experimental import pallas as pl
from jax.experimental.pallas import tpu as pltpu
from jax.experimental.pallas import tpu_sc as plsc

B, C, H, W, K = 64, 8, 128, 128, 128
HW = H * W
N = B * K
NC, NS = 2, 16
NW = NC * NS
IPT = N // NW
NCH = IPT // 16
GPT = C * IPT


def _sc_gather_body(src_hbm, ind_hbm, rb_hbm, rr_hbm,
                    pred_hbm, rb_out_hbm, rr_out_hbm,
                    ind_v, idx_v, idx2_v, rows_v, rb_v, rr_v, sem):
    wid = lax.axis_index("s") * NC + lax.axis_index("c")
    base = wid * IPT
    pltpu.sync_copy(ind_hbm.at[pl.ds(base, IPT)], ind_v)
    copies = []
    two = lax.iota(jnp.int32, 16) * 2
    for h in range(2):
        for chunk in range(NCH):
            idx2_v[pl.ds(h * IPT + chunk * 16, 16)] = (
                two + (2 * base + 32 * chunk + h)
            )
    copies.append(pltpu.async_copy(rb_hbm.at[idx2_v], rb_v, sem))
    copies.append(pltpu.async_copy(rr_hbm.at[idx2_v], rr_v, sem))
    wbase = (wid * (IPT // K)) * (C * HW)
    for half in range(4):
        for c in (2 * half, 2 * half + 1):
            for chunk in range(NCH):
                boff = wbase + (chunk // (K // 16)) * (C * HW)
                idx_v[pl.ds((c * NCH + chunk) * 16, 16)] = (
                    ind_v[pl.ds(chunk * 16, 16)] + (boff + c * HW)
                )
        sl = pl.ds(half * 2 * IPT, 2 * IPT)
        copies.append(
            pltpu.async_copy(src_hbm.at[idx_v.at[sl]], rows_v.at[sl], sem)
        )
    for cp in copies:
        cp.wait()
    pltpu.sync_copy(rows_v, pred_hbm.at[wid])
    pltpu.sync_copy(rb_v, rb_out_hbm.at[wid])
    pltpu.sync_copy(rr_v, rr_out_hbm.at[wid])


@functools.cache
def _sc_gather():
    return functools.partial(
        pl.kernel,
        out_type=(
            jax.ShapeDtypeStruct((NW, GPT), jnp.float32),
            jax.ShapeDtypeStruct((NW, 2 * IPT), jnp.int32),
            jax.ShapeDtypeStruct((NW, 2 * IPT), jnp.float32),
        ),
        mesh=plsc.VectorSubcoreMesh(core_axis_name="c", subcore_axis_name="s"),
        scratch_types=[
            pltpu.VMEM((IPT,), jnp.int32),
            pltpu.VMEM((GPT,), jnp.int32),
            pltpu.VMEM((2 * IPT,), jnp.int32),
            pltpu.VMEM((GPT,), jnp.float32),
            pltpu.VMEM((2 * IPT,), jnp.int32),
            pltpu.VMEM((2 * IPT,), jnp.float32),
            pltpu.SemaphoreType.DMA,
        ],
    )(_sc_gather_body)


def _loss_body(pred_ref, rb_ref, rr_ref, mask_ref, out_ref):
    x = [pred_ref[:, c, :] for c in range(C)]
    mf = mask_ref[...].astype(jnp.float32)
    tb0 = rb_ref[:, 0, :]
    tb1 = rb_ref[:, 1, :]
    tr0 = rr_ref[:, 0, :]
    tr1 = rr_ref[:, 1, :]
    cnt = jnp.sum(mf)

    def pick_logp(a, b, t):
        m = jnp.maximum(a, b)
        lse = m + jnp.log(jnp.exp(a - m) + jnp.exp(b - m))
        return jnp.where(t == 1, b, a) - lse

    s1 = jnp.sum(pick_logp(x[0], x[1], tb0) * mf)
    s2 = jnp.sum(pick_logp(x[4], x[5], tb1) * mf)

    def sl1(p, t):
        d = jnp.abs(p - t)
        return jnp.where(d < 1.0, 0.5 * d * d, d - 0.5)

    w1 = tb0.astype(jnp.float32)
    w2 = tb1.astype(jnp.float32)
    n1 = jnp.sum(w1)
    n2 = jnp.sum(w2)
    r1 = jnp.sum((sl1(x[2], jnp.sin(tr0)) + sl1(x[3], jnp.cos(tr0))) * w1)
    r2 = jnp.sum((sl1(x[6], jnp.sin(tr1)) + sl1(x[7], jnp.cos(tr1))) * w2)

    zero = jnp.float32(0.0)
    lb1 = jnp.where(cnt > 0, -s1 / cnt, zero)
    lb2 = jnp.where(cnt > 0, -s2 / cnt, zero)
    lr = jnp.where(n1 > 0, r1 / n1, zero) + jnp.where(n2 > 0, r2 / n2, zero)
    total = lb1 + lb2 + lr
    out_ref[0, 0] = jnp.where(cnt == 0, zero, total)


_loss = pl.pallas_call(
    _loss_body,
    out_shape=jax.ShapeDtypeStruct((1, 1), jnp.float32),
    out_specs=pl.BlockSpec(memory_space=pltpu.SMEM),
)


def kernel(output, mask, ind, rotbin, rotres):
    src = output.reshape(-1)
    indf = ind.reshape(-1).astype(jnp.int32)
    rbf = rotbin.reshape(-1).astype(jnp.int32)
    rrf = rotres.reshape(-1)
    pred, rb, rr = _sc_gather()(src, indf, rbf, rrf)
    out = _loss(
        pred.reshape(NW, C, IPT),
        rb.reshape(NW, 2, IPT),
        rr.reshape(NW, 2, IPT),
        mask.reshape(NW, IPT).astype(jnp.int32),
    )
    return out[0, 0]


# speedup vs baseline: 1.3686x; 1.3686x over previous
import functools

import jax
import jax.numpy as jnp
from jax import lax
from jax.experimental import pallas as pl
from jax.experimental.pallas import tpu as pltpu
from jax.experimental.pallas import tpu_sc as plsc

B, C, H, W, K = 64, 8, 128, 128, 128
HW = H * W
N = B * K
NC, NS = 2, 16
NW = NC * NS
IPT = N // NW
NCH = IPT // 16
GPT = C * IPT


def _sc_gather_body(src_hbm, ind_hbm, out_hbm, ind_v, idx_v, rows_v, sem):
    wid = lax.axis_index("s") * NC + lax.axis_index("c")
    base = wid * IPT
    pltpu.sync_copy(ind_hbm.at[pl.ds(base, IPT)], ind_v)
    wbase = (wid * (IPT // K)) * (C * HW)
    copies = []
    for half in range(4):
        for c in (2 * half, 2 * half + 1):
            for chunk in range(NCH):
                boff = wbase + (chunk // (K // 16)) * (C * HW)
                idx_v[pl.ds((c * NCH + chunk) * 16, 16)] = (
                    ind_v[pl.ds(chunk * 16, 16)] + (boff + c * HW)
                )
        sl = pl.ds(half * 2 * IPT, 2 * IPT)
        copies.append(
            pltpu.async_copy(src_hbm.at[idx_v.at[sl]], rows_v.at[sl], sem)
        )
    for cp in copies:
        cp.wait()
    pltpu.sync_copy(rows_v, out_hbm.at[wid])


@functools.cache
def _sc_gather():
    return functools.partial(
        pl.kernel,
        out_type=jax.ShapeDtypeStruct((NW, GPT), jnp.float32),
        mesh=plsc.VectorSubcoreMesh(core_axis_name="c", subcore_axis_name="s"),
        scratch_types=[
            pltpu.VMEM((IPT,), jnp.int32),
            pltpu.VMEM((GPT,), jnp.int32),
            pltpu.VMEM((GPT,), jnp.float32),
            pltpu.SemaphoreType.DMA,
        ],
    )(_sc_gather_body)


def _loss_body(pred_ref, mask_ref, tb_ref, tr_ref, out_ref):
    x = [pred_ref[:, c, :] for c in range(C)]
    mf = mask_ref[...].astype(jnp.float32)
    ri = lax.broadcasted_iota(jnp.int32, (2 * IPT, IPT), 0)
    cj = lax.broadcasted_iota(jnp.int32, (2 * IPT, IPT), 1)
    even = (ri == 2 * cj).astype(jnp.float32)
    odd = (ri == 2 * cj + 1).astype(jnp.float32)
    tbf = tb_ref[...].astype(jnp.float32)
    trv = tr_ref[...]
    w1 = jnp.dot(tbf, even)
    w2 = jnp.dot(tbf, odd)
    tr0 = jnp.dot(trv, even, precision=jax.lax.Precision.HIGHEST)
    tr1 = jnp.dot(trv, odd, precision=jax.lax.Precision.HIGHEST)
    cnt = jnp.sum(mf)

    def pick_logp(a, b, t):
        m = jnp.maximum(a, b)
        lse = m + jnp.log(jnp.exp(a - m) + jnp.exp(b - m))
        return jnp.where(t > 0.5, b, a) - lse

    s1 = jnp.sum(pick_logp(x[0], x[1], w1) * mf)
    s2 = jnp.sum(pick_logp(x[4], x[5], w2) * mf)

    def sl1(p, t):
        d = jnp.abs(p - t)
        return jnp.where(d < 1.0, 0.5 * d * d, d - 0.5)

    n1 = jnp.sum(w1)
    n2 = jnp.sum(w2)
    r1 = jnp.sum((sl1(x[2], jnp.sin(tr0)) + sl1(x[3], jnp.cos(tr0))) * w1)
    r2 = jnp.sum((sl1(x[6], jnp.sin(tr1)) + sl1(x[7], jnp.cos(tr1))) * w2)

    zero = jnp.float32(0.0)
    lb1 = jnp.where(cnt > 0, -s1 / cnt, zero)
    lb2 = jnp.where(cnt > 0, -s2 / cnt, zero)
    lr = jnp.where(n1 > 0, r1 / n1, zero) + jnp.where(n2 > 0, r2 / n2, zero)
    total = lb1 + lb2 + lr
    out_ref[0, 0] = jnp.where(cnt == 0, zero, total)


_loss = pl.pallas_call(
    _loss_body,
    out_shape=jax.ShapeDtypeStruct((1, 1), jnp.float32),
    out_specs=pl.BlockSpec(memory_space=pltpu.SMEM),
)


def kernel(output, mask, ind, rotbin, rotres):
    src = output.reshape(-1)
    indf = ind.reshape(-1).astype(jnp.int32)
    pred = _sc_gather()(src, indf).reshape(NW, C, IPT)
    tb = rotbin.reshape(NW, 2 * IPT).astype(jnp.int32)
    tr = rotres.reshape(NW, 2 * IPT)
    out = _loss(pred, mask.reshape(NW, IPT).astype(jnp.int32), tb, tr)
    return out[0, 0]


# speedup vs baseline: 1.5247x; 1.1140x over previous
import functools

import jax
import jax.numpy as jnp
from jax import lax
from jax.experimental import pallas as pl
from jax.experimental.pallas import tpu as pltpu
from jax.experimental.pallas import tpu_sc as plsc

B, C, H, W, K = 64, 8, 128, 128, 128
HW = H * W
N = B * K
NC, NS = 2, 16
NW = NC * NS
IPT = N // NW
NCH = IPT // 16
GPT = C * IPT


def _sc_gather_body(src_hbm, ind_hbm, out_hbm, ind_v, idx_v, rows_v, sem):
    wid = lax.axis_index("s") * NC + lax.axis_index("c")
    base = wid * IPT
    pltpu.sync_copy(ind_hbm.at[pl.ds(base, IPT)], ind_v)
    wbase = (wid * (IPT // K)) * (C * HW)
    copies = []
    for half in range(4):
        for c in (2 * half, 2 * half + 1):
            for chunk in range(NCH):
                boff = wbase + (chunk // (K // 16)) * (C * HW)
                idx_v[pl.ds((c * NCH + chunk) * 16, 16)] = (
                    ind_v[pl.ds(chunk * 16, 16)] + (boff + c * HW)
                )
        sl = pl.ds(half * 2 * IPT, 2 * IPT)
        copies.append(
            pltpu.async_copy(src_hbm.at[idx_v.at[sl]], rows_v.at[sl], sem)
        )
    for cp in copies:
        cp.wait()
    pltpu.sync_copy(rows_v, out_hbm.at[wid])


@functools.cache
def _sc_gather():
    return functools.partial(
        pl.kernel,
        out_type=jax.ShapeDtypeStruct((NW, GPT), jnp.float32),
        mesh=plsc.VectorSubcoreMesh(core_axis_name="c", subcore_axis_name="s"),
        scratch_types=[
            pltpu.VMEM((IPT,), jnp.int32),
            pltpu.VMEM((GPT,), jnp.int32),
            pltpu.VMEM((GPT,), jnp.float32),
            pltpu.SemaphoreType.DMA,
        ],
    )(_sc_gather_body)


def _loss_body(pred_ref, mask_ref, tb_ref, tr_ref, out_ref):
    x = [pred_ref[:, c, :] for c in range(C)]
    mf = mask_ref[...].astype(jnp.float32)
    tb0 = tb_ref[0]
    tb1 = tb_ref[1]
    tr0 = tr_ref[0]
    tr1 = tr_ref[1]
    cnt = jnp.sum(mf)

    def pick_logp(a, b, t):
        m = jnp.maximum(a, b)
        lse = m + jnp.log(jnp.exp(a - m) + jnp.exp(b - m))
        return jnp.where(t == 1, b, a) - lse

    s1 = jnp.sum(pick_logp(x[0], x[1], tb0) * mf)
    s2 = jnp.sum(pick_logp(x[4], x[5], tb1) * mf)

    def sl1(p, t):
        d = jnp.abs(p - t)
        return jnp.where(d < 1.0, 0.5 * d * d, d - 0.5)

    w1 = tb0.astype(jnp.float32)
    w2 = tb1.astype(jnp.float32)
    n1 = jnp.sum(w1)
    n2 = jnp.sum(w2)
    r1 = jnp.sum((sl1(x[2], jnp.sin(tr0)) + sl1(x[3], jnp.cos(tr0))) * w1)
    r2 = jnp.sum((sl1(x[6], jnp.sin(tr1)) + sl1(x[7], jnp.cos(tr1))) * w2)

    zero = jnp.float32(0.0)
    lb1 = jnp.where(cnt > 0, -s1 / cnt, zero)
    lb2 = jnp.where(cnt > 0, -s2 / cnt, zero)
    lr = jnp.where(n1 > 0, r1 / n1, zero) + jnp.where(n2 > 0, r2 / n2, zero)
    total = lb1 + lb2 + lr
    out_ref[0, 0] = jnp.where(cnt == 0, zero, total)


_loss = pl.pallas_call(
    _loss_body,
    out_shape=jax.ShapeDtypeStruct((1, 1), jnp.float32),
    out_specs=pl.BlockSpec(memory_space=pltpu.SMEM),
)


def kernel(output, mask, ind, rotbin, rotres):
    src = output.reshape(-1)
    indf = ind.reshape(-1).astype(jnp.int32)
    pred = _sc_gather()(src, indf).reshape(NW, C, IPT)
    tb = jnp.transpose(rotbin, (2, 0, 1)).reshape(2, NW, IPT).astype(jnp.int32)
    tr = jnp.transpose(rotres, (2, 0, 1)).reshape(2, NW, IPT)
    out = _loss(pred, mask.reshape(NW, IPT).astype(jnp.int32), tb, tr)
    return out[0, 0]
